# baseline (speedup 1.0000x reference)
import functools

import jax
import jax.numpy as jnp
from jax import lax
from jax.experimental import pallas as pl
from jax.experimental.pallas import tpu as pltpu
from jax.experimental.pallas import tpu_sc as plsc

NC = 2
NS = 16
NW = NC * NS
L = 16
GB = 128
CHUNK = 256
NGB = CHUNK // GB
OUT_DIM = 16


def _rsqrt(x):
    i = plsc.bitcast(x, jnp.int32)
    i = jnp.int32(0x5F3759DF) - (i >> 1)
    y = plsc.bitcast(i, jnp.float32)
    for _ in range(2):
        y = y * (1.5 - 0.5 * x * y * y)
    return y


def _cheb(c1, s1):
    c2 = 2.0 * c1 * c1 - 1.0
    s2 = 2.0 * c1 * s1
    c3 = 2.0 * c1 * c2 - c1
    s3 = 2.0 * c1 * s2 - s1
    c4 = 2.0 * c1 * c3 - c2
    s4 = 2.0 * c1 * s3 - s2
    return [c1, c2, c3, c4], [s1, s2, s3, s4]


def _sc_body(total_chunks, src_hbm, dst_hbm, comb_hbm, out_hbm,
             sidx, didx, sbuf, dbuf, obuf,
             sem_i0, sem_i1, sem_g0, sem_g1, sem_o0, sem_o1):
    wid = lax.axis_index("s") * NC + lax.axis_index("c")
    n = (total_chunks - wid + NW - 1) // NW
    iota = lax.iota(jnp.int32, L)

    idx_sems = (sem_i0, sem_i1)
    g_sems = (sem_g0, sem_g1)
    o_sems = (sem_o0, sem_o1)

    def fire_idx(c, b):
        base = (wid + c * NW) * CHUNK
        pltpu.async_copy(src_hbm.at[pl.ds(base, CHUNK)], sidx.at[b], idx_sems[b])
        pltpu.async_copy(dst_hbm.at[pl.ds(base, CHUNK)], didx.at[b], idx_sems[b])

    def wait_idx(b):
        pltpu.make_async_copy(src_hbm.at[pl.ds(0, CHUNK)], sidx.at[b],
                              idx_sems[b]).wait()
        pltpu.make_async_copy(dst_hbm.at[pl.ds(0, CHUNK)], didx.at[b],
                              idx_sems[b]).wait()

    def fire_gathers(b):
        for j in range(NGB):
            rows = pl.ds(j * GB, GB)
            pltpu.async_copy(comb_hbm.at[sidx.at[b, rows]],
                             sbuf.at[b, rows], g_sems[b])
            pltpu.async_copy(comb_hbm.at[didx.at[b, rows]],
                             dbuf.at[b, rows], g_sems[b])

    def wait_gathers(b):
        for j in range(NGB):
            rows = pl.ds(j * GB, GB)
            pltpu.make_async_copy(comb_hbm.at[sidx.at[b, rows]],
                                  sbuf.at[b, rows], g_sems[b]).wait()
            pltpu.make_async_copy(comb_hbm.at[didx.at[b, rows]],
                                  dbuf.at[b, rows], g_sems[b]).wait()

    def fire_out(c, b):
        base = (wid + c * NW) * CHUNK
        pltpu.async_copy(obuf.at[b], out_hbm.at[pl.ds(base, CHUNK)],
                         o_sems[b])

    def wait_out(b):
        pltpu.make_async_copy(obuf.at[b], out_hbm.at[pl.ds(0, CHUNK)],
                              o_sems[b]).wait()

    def compute(b):
        for g in range(CHUNK // L):
            row = iota + (g * L)


            def ld(buf, col):
                cvec = jnp.full((L,), col, dtype=jnp.int32)
                return plsc.load_gather(buf.at[b], [row, cvec])

            sx = ld(sbuf, 0); sy = ld(sbuf, 1); sz = ld(sbuf, 2)
            dx = ld(dbuf, 0); dy = ld(dbuf, 1); dz = ld(dbuf, 2)
            f00 = ld(dbuf, 3); f01 = ld(dbuf, 4); f02 = ld(dbuf, 5)
            f10 = ld(dbuf, 6); f11 = ld(dbuf, 7); f12 = ld(dbuf, 8)
            f20 = ld(dbuf, 9); f21 = ld(dbuf, 10); f22 = ld(dbuf, 11)

            rx = sx - dx
            ry = sy - dy
            rz = sz - dz
            v0 = f00 * rx + f01 * ry + f02 * rz
            v1 = f10 * rx + f11 * ry + f12 * rz
            v2 = f20 * rx + f21 * ry + f22 * rz

            rho2 = v0 * v0 + v1 * v1
            r2 = rho2 + v2 * v2
            cz = v2 * _rsqrt(r2)
            cz = jnp.minimum(jnp.maximum(cz, -1.0 + 1e-7), 1.0 - 1e-7)
            t1 = 1.0 - cz * cz
            st = t1 * _rsqrt(t1)

            invrho = _rsqrt(rho2)
            pos_rho = rho2 > 0.0
            cp = jnp.where(pos_rho, v0 * invrho, 1.0)
            sp = jnp.where(pos_rho, v1 * invrho, 0.0)

            ct_k, st_k = _cheb(cz, st)
            cp_k, sp_k = _cheb(cp, sp)
            feats = ct_k + st_k + cp_k + sp_k
            for f in range(OUT_DIM):
                cvec = jnp.full((L,), f, dtype=jnp.int32)
                plsc.store_scatter(obuf.at[b], [row, cvec], feats[f])

    fire_idx(0, 0)
    fire_idx(1, 1)
    wait_idx(0)
    fire_gathers(0)

    def pair_body(m, carry):
        for b in (0, 1):
            c = 2 * m + b
            nxt = c + 1

            @pl.when(c < n)
            def _():
                wait_gathers(b)

                @pl.when(c + 2 < n)
                def _():
                    fire_idx(c + 2, b)

                @pl.when(nxt < n)
                def _():
                    wait_idx(1 - b)
                    fire_gathers(1 - b)

                @pl.when(c >= 2)
                def _():
                    wait_out(b)

                compute(b)

                @pl.when(c >= 1)
                def _():
                    fire_out(c - 1, 1 - b)
        return carry

    lax.fori_loop(0, (n + 1) // 2, pair_body, 0)
    last = n - 1

    @pl.when(last % 2 == 0)
    def _():
        fire_out(last, 0)

    @pl.when(last % 2 == 1)
    def _():
        fire_out(last, 1)

    wait_out(0)
    wait_out(1)


def kernel(pos, edge_index, lframes):
    n = pos.shape[0]
    e = edge_index.shape[1]
    assert e % CHUNK == 0
    total_chunks = e // CHUNK
    src = edge_index[0]
    dst = edge_index[1]
    comb = jnp.concatenate(
        [pos, lframes.reshape(n, 9), jnp.zeros((n, 4), jnp.float32)], axis=1)

    mesh = plsc.VectorSubcoreMesh(core_axis_name="c", subcore_axis_name="s")
    run = pl.kernel(
        functools.partial(_sc_body, total_chunks),
        out_type=jax.ShapeDtypeStruct((e, OUT_DIM), jnp.float32),
        mesh=mesh,
        compiler_params=pltpu.CompilerParams(
            needs_layout_passes=False, use_tc_tiling_on_sc=False),
        scratch_types=[
            pltpu.VMEM((2, CHUNK), jnp.int32),
            pltpu.VMEM((2, CHUNK), jnp.int32),
            pltpu.VMEM((2, CHUNK, OUT_DIM), jnp.float32),
            pltpu.VMEM((2, CHUNK, OUT_DIM), jnp.float32),
            pltpu.VMEM((2, CHUNK, OUT_DIM), jnp.float32),
            pltpu.SemaphoreType.DMA,
            pltpu.SemaphoreType.DMA,
            pltpu.SemaphoreType.DMA,
            pltpu.SemaphoreType.DMA,
            pltpu.SemaphoreType.DMA,
            pltpu.SemaphoreType.DMA,
        ],
    )
    return run(src, dst, comb)


# speedup vs baseline: 1.3718x; 1.3718x over previous
import functools

import jax
import jax.numpy as jnp
from jax import lax
from jax.experimental import pallas as pl
from jax.experimental.pallas import tpu as pltpu
from jax.experimental.pallas import tpu_sc as plsc

NC = 2
NS = 16
NW = NC * NS
L = 16
GB = 128
CHUNK = 256
NGB = CHUNK // GB
OUT_DIM = 16


def _rsqrt(x):
    i = plsc.bitcast(x, jnp.int32)
    i = jnp.int32(0x5F3759DF) - (i >> 1)
    y = plsc.bitcast(i, jnp.float32)
    for _ in range(2):
        y = y * (1.5 - 0.5 * x * y * y)
    return y


def _cheb(c1, s1):
    c2 = 2.0 * c1 * c1 - 1.0
    s2 = 2.0 * c1 * s1
    c3 = 2.0 * c1 * c2 - c1
    s3 = 2.0 * c1 * s2 - s1
    c4 = 2.0 * c1 * c3 - c2
    s4 = 2.0 * c1 * s3 - s2
    return [c1, c2, c3, c4], [s1, s2, s3, s4]


def _sc_body(total_chunks, src_hbm, dst_hbm, comb_hbm, out_hbm,
             sidx, didx, sbuf, dbuf, obuf,
             sem_i0, sem_i1, sem_g0, sem_g1, sem_o0, sem_o1):
    wid = lax.axis_index("s") * NC + lax.axis_index("c")
    n = (total_chunks - wid + NW - 1) // NW
    iota = lax.iota(jnp.int32, L)

    idx_sems = (sem_i0, sem_i1)
    g_sems = (sem_g0, sem_g1)
    o_sems = (sem_o0, sem_o1)

    def fire_idx(c, b):
        base = (wid + c * NW) * CHUNK
        pltpu.async_copy(src_hbm.at[pl.ds(base, CHUNK)], sidx.at[b], idx_sems[b])
        pltpu.async_copy(dst_hbm.at[pl.ds(base, CHUNK)], didx.at[b], idx_sems[b])

    def wait_idx(b):
        pltpu.make_async_copy(src_hbm.at[pl.ds(0, CHUNK)], sidx.at[b],
                              idx_sems[b]).wait()
        pltpu.make_async_copy(dst_hbm.at[pl.ds(0, CHUNK)], didx.at[b],
                              idx_sems[b]).wait()

    def fire_gathers(b):
        for j in range(NGB):
            rows = pl.ds(j * GB, GB)
            pltpu.async_copy(comb_hbm.at[sidx.at[b, rows]],
                             sbuf.at[b, rows], g_sems[b])
            pltpu.async_copy(comb_hbm.at[didx.at[b, rows]],
                             dbuf.at[b, rows], g_sems[b])

    def wait_gathers(b):
        for j in range(NGB):
            rows = pl.ds(j * GB, GB)
            pltpu.make_async_copy(comb_hbm.at[sidx.at[b, rows]],
                                  sbuf.at[b, rows], g_sems[b]).wait()
            pltpu.make_async_copy(comb_hbm.at[didx.at[b, rows]],
                                  dbuf.at[b, rows], g_sems[b]).wait()

    def fire_out(c, b):
        base = (wid + c * NW) * CHUNK
        pltpu.async_copy(obuf.at[b], out_hbm.at[pl.ds(base, CHUNK)],
                         o_sems[b])

    def wait_out(b):
        pltpu.make_async_copy(obuf.at[b], out_hbm.at[pl.ds(0, CHUNK)],
                              o_sems[b]).wait()

    def compute(b):
        for g in range(CHUNK // L):
            row = iota + (g * L)


            def ld(buf, col):
                cvec = jnp.full((L,), col, dtype=jnp.int32)
                return plsc.load_gather(buf.at[b], [row, cvec])

            sx = ld(sbuf, 0); sy = ld(sbuf, 1); sz = ld(sbuf, 2)
            dx = ld(dbuf, 0); dy = ld(dbuf, 1); dz = ld(dbuf, 2)
            f00 = ld(dbuf, 3); f01 = ld(dbuf, 4); f02 = ld(dbuf, 5)
            f10 = ld(dbuf, 6); f11 = ld(dbuf, 7); f12 = ld(dbuf, 8)
            f20 = ld(dbuf, 9); f21 = ld(dbuf, 10); f22 = ld(dbuf, 11)

            rx = sx - dx
            ry = sy - dy
            rz = sz - dz
            v0 = f00 * rx + f01 * ry + f02 * rz
            v1 = f10 * rx + f11 * ry + f12 * rz
            v2 = f20 * rx + f21 * ry + f22 * rz

            rho2 = v0 * v0 + v1 * v1
            r2 = rho2 + v2 * v2
            cz = v2 * _rsqrt(r2)
            cz = jnp.minimum(jnp.maximum(cz, -1.0 + 1e-7), 1.0 - 1e-7)
            t1 = 1.0 - cz * cz
            st = t1 * _rsqrt(t1)

            invrho = _rsqrt(rho2)
            pos_rho = rho2 > 0.0
            cp = jnp.where(pos_rho, v0 * invrho, 1.0)
            sp = jnp.where(pos_rho, v1 * invrho, 0.0)

            ct_k, st_k = _cheb(cz, st)
            cp_k, sp_k = _cheb(cp, sp)
            feats = ct_k + st_k + cp_k + sp_k
            for f in range(OUT_DIM):
                cvec = jnp.full((L,), f, dtype=jnp.int32)
                plsc.store_scatter(obuf.at[b], [row, cvec], feats[f])

    fire_idx(0, 0)
    fire_idx(1, 1)
    wait_idx(0)
    fire_gathers(0)

    def pair_body(m, carry):
        for b in (0, 1):
            c = 2 * m + b
            nxt = c + 1

            @pl.when(c < n)
            def _():
                wait_gathers(b)

                @pl.when(c + 2 < n)
                def _():
                    fire_idx(c + 2, b)

                @pl.when(nxt < n)
                def _():
                    wait_idx(1 - b)
                    fire_gathers(1 - b)

                @pl.when(c >= 2)
                def _():
                    wait_out(b)

                compute(b)

                @pl.when(c >= 1)
                def _():
                    fire_out(c - 1, 1 - b)
        return carry

    lax.fori_loop(0, (n + 1) // 2, pair_body, 0)
    last = n - 1

    @pl.when(last % 2 == 0)
    def _():
        fire_out(last, 0)

    @pl.when(last % 2 == 1)
    def _():
        fire_out(last, 1)

    wait_out(0)
    wait_out(1)


def kernel(pos, edge_index, lframes):
    n = pos.shape[0]
    e = edge_index.shape[1]
    src = edge_index[0]
    dst = edge_index[1]
    comb = jnp.concatenate(
        [pos, lframes.reshape(n, 9), jnp.zeros((n, 4), jnp.float32)], axis=1)

    mesh = plsc.VectorSubcoreMesh(core_axis_name="c", subcore_axis_name="s")

    nparts = 5
    if e % (nparts * CHUNK) != 0:
        nparts = 1
    ep = e // nparts
    total_chunks = ep // CHUNK
    run = pl.kernel(
        functools.partial(_sc_body, total_chunks),
        out_type=jax.ShapeDtypeStruct((ep, OUT_DIM), jnp.float32),
        mesh=mesh,
        compiler_params=pltpu.CompilerParams(
            needs_layout_passes=False, use_tc_tiling_on_sc=False),
        scratch_types=[
            pltpu.VMEM((2, CHUNK), jnp.int32),
            pltpu.VMEM((2, CHUNK), jnp.int32),
            pltpu.VMEM((2, CHUNK, OUT_DIM), jnp.float32),
            pltpu.VMEM((2, CHUNK, OUT_DIM), jnp.float32),
            pltpu.VMEM((2, CHUNK, OUT_DIM), jnp.float32),
            pltpu.SemaphoreType.DMA,
            pltpu.SemaphoreType.DMA,
            pltpu.SemaphoreType.DMA,
            pltpu.SemaphoreType.DMA,
            pltpu.SemaphoreType.DMA,
            pltpu.SemaphoreType.DMA,
        ],
    )
    outs = [run(lax.slice_in_dim(src, k * ep, (k + 1) * ep),
                lax.slice_in_dim(dst, k * ep, (k + 1) * ep), comb)
            for k in range(nparts)]
    if nparts == 1:
        return outs[0]
    return jnp.concatenate(outs, axis=0)
